# baseline (speedup 1.0000x reference)
import jax
import jax.numpy as jnp
from jax import lax
from jax.experimental import pallas as pl
from jax.experimental.pallas import tpu as pltpu
from jax.experimental.pallas import tpu_sc as plsc

N = 10000
E = 320000
C = 128
CE = 144

NS = 16
LANES = 16
PER_TILE = E // NS
CHUNK = 80
NCHUNK = PER_TILE // CHUNK
BCH = 25
ROWS_MAIN = 624
ZROWS = 16

_SC_PARAMS = pltpu.CompilerParams(needs_layout_passes=False,
                                  use_tc_tiling_on_sc=False)


def _zero_init(acc_sh, zsrc, sid, sem, width):
    zv = jnp.zeros((LANES,), jnp.float32)

    def zrow(i, _):
        for r in range(width // LANES):
            zsrc[i, pl.ds(r * LANES, LANES)] = zv
        return 0
    lax.fori_loop(0, ZROWS, zrow, 0)

    r0 = sid * ROWS_MAIN
    n_z = jnp.where(sid == NS - 1, (N - (NS - 1) * ROWS_MAIN) // ZROWS,
                    ROWS_MAIN // ZROWS)

    zslice = zsrc.at[pl.ds(0, ZROWS)]

    def zcopy(j, _):
        pltpu.async_copy(zslice, acc_sh.at[pl.ds(r0 + j * ZROWS, ZROWS)], sem)
        return 0
    lax.fori_loop(0, n_z, zcopy, 0)

    def zdrain(j, _):
        pltpu.make_async_copy(zslice, acc_sh.at[pl.ds(r0, ZROWS)], sem).wait()
        return 0
    lax.fori_loop(0, n_z, zdrain, 0)
    return r0, n_z


def _writeback(acc_sh, out_hbm, r0, n_z, sem):
    def wcopy(j, _):
        pltpu.async_copy(acc_sh.at[pl.ds(r0 + j * ZROWS, ZROWS)],
                         out_hbm.at[pl.ds(r0 + j * ZROWS, ZROWS)], sem)
        return 0
    lax.fori_loop(0, n_z, wcopy, 0)

    def wdrain(j, _):
        pltpu.make_async_copy(acc_sh.at[pl.ds(r0, ZROWS)],
                              out_hbm.at[pl.ds(r0, ZROWS)], sem).wait()
        return 0
    lax.fori_loop(0, n_z, wdrain, 0)


def _edge_pipeline(h_hbm, s2, d2, acc_sh, sblk, dblk, rows_a, rows_b,
                   sem_ga, sem_gb, sid, process):
    row0 = sid * (PER_TILE // CHUNK)

    def refill(b):
        pltpu.sync_copy(s2.at[pl.ds(row0 + b * BCH, BCH)], sblk)
        pltpu.sync_copy(d2.at[pl.ds(row0 + b * BCH, BCH)], dblk)

    refill(0)
    pltpu.async_copy(h_hbm.at[sblk.at[0]], rows_a, sem_ga)

    def iter_body(c, _):
        def half(buf, sem_g, obuf, osem_g):
            jr = lax.rem(c, BCH)
            nxt = c + 1
            jn = lax.rem(nxt, BCH)
            pltpu.make_async_copy(h_hbm.at[sblk.at[jr]], buf, sem_g).wait()

            @pl.when((nxt < NCHUNK) & (jn != 0))
            def _():
                pltpu.async_copy(h_hbm.at[sblk.at[jn]], obuf, osem_g)

            process(buf, jr)
            pltpu.sync_copy(buf, acc_sh.at[dblk.at[jr]], add=True)

            @pl.when((nxt < NCHUNK) & (jn == 0))
            def _():
                refill(nxt // BCH)
                pltpu.async_copy(h_hbm.at[sblk.at[0]], obuf, osem_g)

        @pl.when(lax.rem(c, 2) == 0)
        def _():
            half(rows_a, sem_ga, rows_b, sem_gb)

        @pl.when(lax.rem(c, 2) == 1)
        def _():
            half(rows_b, sem_gb, rows_a, sem_ga)
        return 0
    lax.fori_loop(0, NCHUNK, iter_body, 0)


def _edge_pipeline3(h_hbm, s2, d2, acc_sh, sblk, dblk, bufs, sems, sid):
    row0 = sid * (PER_TILE // CHUNK)

    def refill(b):
        pltpu.sync_copy(s2.at[pl.ds(row0 + b * BCH, BCH)], sblk)
        pltpu.sync_copy(d2.at[pl.ds(row0 + b * BCH, BCH)], dblk)

    refill(0)
    pltpu.async_copy(h_hbm.at[sblk.at[0]], bufs[0], sems[0])

    def iter_body(c, _):
        def rot(buf, sem, buf1, sem1, buf2, sem2):
            jr = lax.rem(c, BCH)
            nxt = c + 1
            jn = lax.rem(nxt, BCH)
            pltpu.make_async_copy(h_hbm.at[sblk.at[jr]], buf, sem).wait()

            @pl.when(jr == 0)
            def _():
                pltpu.async_copy(h_hbm.at[sblk.at[1]], buf1, sem1)

            @pl.when((c + 2 < NCHUNK) & (jr <= BCH - 3))
            def _():
                pltpu.async_copy(h_hbm.at[sblk.at[jr + 2]], buf2, sem2)

            pltpu.sync_copy(buf, acc_sh.at[dblk.at[jr]], add=True)

            @pl.when((nxt < NCHUNK) & (jn == 0))
            def _():
                refill(nxt // BCH)
                pltpu.async_copy(h_hbm.at[sblk.at[0]], buf1, sem1)

        for par in range(3):
            @pl.when(lax.rem(c, 3) == par)
            def _(par=par):
                rot(bufs[par], sems[par],
                    bufs[(par + 1) % 3], sems[(par + 1) % 3],
                    bufs[(par + 2) % 3], sems[(par + 2) % 3])
        return 0
    lax.fori_loop(0, NCHUNK, iter_body, 0)


def _sage_sc_body(h_u2i, h_i2u, su2, du2, si2, di2, out_item, out_user,
                  acc_sh, sblk, dblk, rows_a, rows_b, rows_c,
                  sem_ga, sem_gb, sem_gc):
    cid = lax.axis_index("c")
    sid = lax.axis_index("s")

    r0, n_z = _zero_init(acc_sh, rows_a, sid, sem_ga, C)
    plsc.subcore_barrier()

    def run_rel(h_hbm, s2, d2, out_hbm):
        _edge_pipeline3(h_hbm, s2, d2, acc_sh, sblk, dblk,
                        (rows_a, rows_b, rows_c), (sem_ga, sem_gb, sem_gc), sid)
        plsc.subcore_barrier()
        _writeback(acc_sh, out_hbm, r0, n_z, sem_ga)

    @pl.when(cid == 0)
    def _():
        run_rel(h_u2i, su2, du2, out_item)

    @pl.when(cid == 1)
    def _():
        run_rel(h_i2u, si2, di2, out_user)


_sage_sc = pl.kernel(
    _sage_sc_body,
    out_type=(jax.ShapeDtypeStruct((N, C), jnp.float32),
              jax.ShapeDtypeStruct((N, C), jnp.float32)),
    mesh=plsc.VectorSubcoreMesh(core_axis_name="c", subcore_axis_name="s"),
    compiler_params=_SC_PARAMS,
    scratch_types=[
        pltpu.VMEM_SHARED((N, C), jnp.float32),
        pltpu.VMEM((BCH, CHUNK), jnp.int32),
        pltpu.VMEM((BCH, CHUNK), jnp.int32),
        pltpu.VMEM((CHUNK, C), jnp.float32),
        pltpu.VMEM((CHUNK, C), jnp.float32),
        pltpu.VMEM((CHUNK, C), jnp.float32),
        pltpu.SemaphoreType.DMA,
        pltpu.SemaphoreType.DMA,
        pltpu.SemaphoreType.DMA,
    ],
)


def _gat_sc_body(hsx_u2i, sdst_u2i, smax_u2i, su2, du2,
                 hsx_i2u, sdst_i2u, smax_i2u, si2, di2,
                 out_item, out_user,
                 acc_sh, sdst_v, smax_v, w_v, sblk, dblk, rows_a, rows_b,
                 sem_ga, sem_gb):
    cid = lax.axis_index("c")
    sid = lax.axis_index("s")

    r0, n_z = _zero_init(acc_sh, rows_a, sid, sem_ga, CE)
    plsc.subcore_barrier()

    lanes = lax.iota(jnp.int32, LANES)

    def run_rel(hsx_hbm, sdst_hbm, smax_hbm, s2, d2, out_hbm):
        pltpu.sync_copy(sdst_hbm, sdst_v)
        pltpu.sync_copy(smax_hbm, smax_v)

        acc = smax_v[pl.ds(0, LANES)]
        for i in range(1, 80 // LANES):
            acc = jnp.maximum(acc, smax_v[pl.ds(i * LANES, LANES)])
        for k in (8, 4, 2, 1):
            w_v[pl.ds(0, LANES)] = acc
            acc = jnp.maximum(acc, plsc.load_gather(w_v, [lanes ^ k]))
        smax = acc

        col129 = jnp.full((LANES,), C + 1, jnp.int32)

        def process(buf, jr):
            for g in range(CHUNK // LANES):
                e16 = lanes + g * LANES
                d16 = dblk[jr, pl.ds(g * LANES, LANES)]
                a = plsc.load_gather(buf, [e16, col129])
                b = plsc.load_gather(sdst_v, [d16])
                x = a + b
                l = jnp.where(x >= 0, x, 0.2 * x)
                y = smax + b
                m = jnp.where(y >= 0, y, 0.2 * y)
                w_v[pl.ds(g * LANES, LANES)] = jnp.exp(l - m)

            @plsc.parallel_loop(0, CHUNK, step=1, unroll=4)
            def _(e):
                wspl = plsc.load_gather(w_v, [jnp.full((LANES,), e, jnp.int32)])
                for r in range(CE // LANES):
                    buf[e, pl.ds(r * LANES, LANES)] = buf[e, pl.ds(r * LANES, LANES)] * wspl

        _edge_pipeline(hsx_hbm, s2, d2, acc_sh, sblk, dblk, rows_a, rows_b,
                       sem_ga, sem_gb, sid, process)
        plsc.subcore_barrier()
        _writeback(acc_sh, out_hbm, r0, n_z, sem_ga)

    @pl.when(cid == 0)
    def _():
        run_rel(hsx_u2i, sdst_u2i, smax_u2i, su2, du2, out_item)

    @pl.when(cid == 1)
    def _():
        run_rel(hsx_i2u, sdst_i2u, smax_i2u, si2, di2, out_user)


_gat_sc = pl.kernel(
    _gat_sc_body,
    out_type=(jax.ShapeDtypeStruct((N, CE), jnp.float32),
              jax.ShapeDtypeStruct((N, CE), jnp.float32)),
    mesh=plsc.VectorSubcoreMesh(core_axis_name="c", subcore_axis_name="s"),
    compiler_params=_SC_PARAMS,
    scratch_types=[
        pltpu.VMEM_SHARED((N, CE), jnp.float32),
        pltpu.VMEM((N,), jnp.float32),
        pltpu.VMEM((80,), jnp.float32),
        pltpu.VMEM((CHUNK,), jnp.float32),
        pltpu.VMEM((BCH, CHUNK), jnp.int32),
        pltpu.VMEM((BCH, CHUNK), jnp.int32),
        pltpu.VMEM((CHUNK, CE), jnp.float32),
        pltpu.VMEM((CHUNK, CE), jnp.float32),
        pltpu.SemaphoreType.DMA,
        pltpu.SemaphoreType.DMA,
    ],
)


def _ln(x, w, b):
    mu = x.mean(-1, keepdims=True)
    var = ((x - mu) ** 2).mean(-1, keepdims=True)
    return (x - mu) / jnp.sqrt(var + 1e-5) * w + b


BLK = 1000
_full = pl.BlockSpec((128, 128), lambda i: (0, 0))
_vec = pl.BlockSpec((1, 128), lambda i: (0, 0))
_rows = pl.BlockSpec((BLK, C), lambda i: (i, 0))
_rows_e = pl.BlockSpec((BLK, CE), lambda i: (i, 0))
_vec8 = pl.BlockSpec((128, 8), lambda i: (0, 0))
_blk8 = pl.BlockSpec((1, 1, 8), lambda i: (i, 0, 0))


def _pre_h_body(xu, xi, wl_u, wl_i, h_u, h_i):
    h_u[...] = jnp.dot(xu[...], wl_u[...], preferred_element_type=jnp.float32)
    h_i[...] = jnp.dot(xi[...], wl_i[...], preferred_element_type=jnp.float32)


def _pre_h_tc(xu, xi, wl_u, wl_i):
    return pl.pallas_call(
        _pre_h_body,
        grid=(N // BLK,),
        in_specs=[_rows, _rows, _full, _full],
        out_specs=[_rows, _rows],
        out_shape=[jax.ShapeDtypeStruct((N, C), jnp.float32)] * 2,
    )(xu, xi, wl_u, wl_i)


def _mid_body(acc_i, acc_u, xu, xi, wr_u, wr_i, bl_u, bl_i,
              n0iw, n0ib, n0uw, n0ub,
              wsrc_u, wdst_u, asrc_u, adst_u,
              wsrc_i, wdst_i, asrc_i, adst_i,
              hsx_u, hsx_i, sdst_u, sdst_i, smax_u, smax_i):
    r_i = jnp.dot(xi[...], wr_u[...], preferred_element_type=jnp.float32) + bl_u[...]
    r_u = jnp.dot(xu[...], wr_i[...], preferred_element_type=jnp.float32) + bl_i[...]
    item1 = jax.nn.relu(_ln(acc_i[...] + r_i, n0iw[...], n0ib[...]))
    user1 = jax.nn.relu(_ln(acc_u[...] + r_u, n0uw[...], n0ub[...]))

    def gat_half(xs, xd, wsrc, wdst, asrc, adst, hsx, sdst, smax):
        hs = jnp.dot(xs, wsrc[...], preferred_element_type=jnp.float32)
        hd = jnp.dot(xd, wdst[...], preferred_element_type=jnp.float32)
        ssrc8 = jnp.dot(hs, asrc[...], preferred_element_type=jnp.float32)
        hsx[...] = jnp.concatenate(
            [hs, jnp.ones((BLK, 1), jnp.float32), ssrc8[:, :1],
             jnp.zeros((BLK, CE - C - 2), jnp.float32)], axis=1)
        sdst[...] = jnp.dot(hd, adst[...], preferred_element_type=jnp.float32)[None, :, :1]
        smax[...] = jnp.broadcast_to(jnp.max(ssrc8[:, 0]), (1, 1, 8))

    gat_half(user1, item1, wsrc_u, wdst_u, asrc_u, adst_u, hsx_u, sdst_u, smax_u)
    gat_half(item1, user1, wsrc_i, wdst_i, asrc_i, adst_i, hsx_i, sdst_i, smax_i)


_sd1 = pl.BlockSpec((1, BLK, 1), lambda i: (i, 0, 0))


def _mid_tc(acc_i, acc_u, xu, xi, wr_u, wr_i, bl_u, bl_i, n0,
            wsrc_u, wdst_u, asrc_u, adst_u,
            wsrc_i, wdst_i, asrc_i, adst_i):
    return pl.pallas_call(
        _mid_body,
        grid=(N // BLK,),
        in_specs=[_rows, _rows, _rows, _rows, _full, _full, _vec, _vec,
                  _vec, _vec, _vec, _vec,
                  _full, _full, _vec8, _vec8,
                  _full, _full, _vec8, _vec8],
        out_specs=[_rows_e, _rows_e, _sd1, _sd1, _blk8, _blk8],
        out_shape=[jax.ShapeDtypeStruct((N, CE), jnp.float32),
                   jax.ShapeDtypeStruct((N, CE), jnp.float32),
                   jax.ShapeDtypeStruct((N // BLK, BLK, 1), jnp.float32),
                   jax.ShapeDtypeStruct((N // BLK, BLK, 1), jnp.float32),
                   jax.ShapeDtypeStruct((N // BLK, 1, 8), jnp.float32),
                   jax.ShapeDtypeStruct((N // BLK, 1, 8), jnp.float32)],
    )(acc_i, acc_u, xu, xi, wr_u, wr_i, bl_u, bl_i, *n0,
      wsrc_u, wdst_u, asrc_u, adst_u,
      wsrc_i, wdst_i, asrc_i, adst_i)


def _post_gat_body(acc_gi, acc_gu, b_u, b_i, n1iw, n1ib, n1uw, n1ub,
                   item2, user2):
    def fin(acc, b, w, bb):
        num = acc[:, :C]
        den = acc[:, C:C + 1]
        return jax.nn.relu(_ln(num / (den + 1e-16) + b[...], w[...], bb[...]))
    item2[...] = fin(acc_gi[...], b_u, n1iw, n1ib)
    user2[...] = fin(acc_gu[...], b_i, n1uw, n1ub)


def _post_gat_tc(acc_gi, acc_gu, b_u, b_i, n1):
    return pl.pallas_call(
        _post_gat_body,
        grid=(N // BLK,),
        in_specs=[_rows_e, _rows_e, _vec, _vec, _vec, _vec, _vec, _vec],
        out_specs=[_rows, _rows],
        out_shape=[jax.ShapeDtypeStruct((N, C), jnp.float32),
                   jax.ShapeDtypeStruct((N, C), jnp.float32)],
    )(acc_gi, acc_gu, b_u, b_i, *n1)


def kernel(x_user, x_item, edge_index_u2i, edge_index_i2u, params):
    p = params
    su2 = jnp.asarray(edge_index_u2i[0], jnp.int32).reshape(E // CHUNK, CHUNK)
    du2 = jnp.asarray(edge_index_u2i[1], jnp.int32).reshape(E // CHUNK, CHUNK)
    si2 = jnp.asarray(edge_index_i2u[0], jnp.int32).reshape(E // CHUNK, CHUNK)
    di2 = jnp.asarray(edge_index_i2u[1], jnp.int32).reshape(E // CHUNK, CHUNK)

    h_u2i, h_i2u = _pre_h_tc(x_user, x_item, p['sage_u2i_Wl'], p['sage_i2u_Wl'])
    acc_item, acc_user = _sage_sc(h_u2i, h_i2u, su2, du2, si2, di2)

    n0 = (p['norm0_item_w'][None, :], p['norm0_item_b'][None, :],
          p['norm0_user_w'][None, :], p['norm0_user_b'][None, :])
    hsx_u, hsx_i, sdst_u2, sdst_i2, smax_u, smax_i = _mid_tc(
        acc_item, acc_user, x_user, x_item,
        p['sage_u2i_Wr'], p['sage_i2u_Wr'],
        p['sage_u2i_bl'][None, :], p['sage_i2u_bl'][None, :], n0,
        p['gat_u2i_Wsrc'], p['gat_u2i_Wdst'],
        jnp.broadcast_to(p['gat_u2i_asrc'][:, None], (C, 8)),
        jnp.broadcast_to(p['gat_u2i_adst'][:, None], (C, 8)),
        p['gat_i2u_Wsrc'], p['gat_i2u_Wdst'],
        jnp.broadcast_to(p['gat_i2u_asrc'][:, None], (C, 8)),
        jnp.broadcast_to(p['gat_i2u_adst'][:, None], (C, 8)))

    accg_item, accg_user = _gat_sc(
        hsx_u, sdst_u2.reshape(N), smax_u.reshape(80), su2, du2,
        hsx_i, sdst_i2.reshape(N), smax_i.reshape(80), si2, di2)

    n1 = (p['norm1_item_w'][None, :], p['norm1_item_b'][None, :],
          p['norm1_user_w'][None, :], p['norm1_user_b'][None, :])
    item2, user2 = _post_gat_tc(accg_item, accg_user,
                                p['gat_u2i_b'][None, :], p['gat_i2u_b'][None, :], n1)
    return (user2, item2)


# speedup vs baseline: 1.0338x; 1.0338x over previous
import jax
import jax.numpy as jnp
from jax import lax
from jax.experimental import pallas as pl
from jax.experimental.pallas import tpu as pltpu
from jax.experimental.pallas import tpu_sc as plsc

N = 10000
E = 320000
C = 128
CE = 144

NS = 16
LANES = 16
PER_TILE = E // NS
CHUNK = 80
NCHUNK = PER_TILE // CHUNK
BCH = 25
ROWS_MAIN = 624
ZROWS = 16

_SC_PARAMS = pltpu.CompilerParams(needs_layout_passes=False,
                                  use_tc_tiling_on_sc=False)


def _zero_init(acc_sh, zsrc, sid, sem, width):
    zv = jnp.zeros((LANES,), jnp.float32)

    def zrow(i, _):
        for r in range(width // LANES):
            zsrc[i, pl.ds(r * LANES, LANES)] = zv
        return 0
    lax.fori_loop(0, ZROWS, zrow, 0)

    r0 = sid * ROWS_MAIN
    n_z = jnp.where(sid == NS - 1, (N - (NS - 1) * ROWS_MAIN) // ZROWS,
                    ROWS_MAIN // ZROWS)

    zslice = zsrc.at[pl.ds(0, ZROWS)]

    def zcopy(j, _):
        pltpu.async_copy(zslice, acc_sh.at[pl.ds(r0 + j * ZROWS, ZROWS)], sem)
        return 0
    lax.fori_loop(0, n_z, zcopy, 0)

    def zdrain(j, _):
        pltpu.make_async_copy(zslice, acc_sh.at[pl.ds(r0, ZROWS)], sem).wait()
        return 0
    lax.fori_loop(0, n_z, zdrain, 0)
    return r0, n_z


def _writeback(acc_sh, out_hbm, r0, n_z, sem):
    def wcopy(j, _):
        pltpu.async_copy(acc_sh.at[pl.ds(r0 + j * ZROWS, ZROWS)],
                         out_hbm.at[pl.ds(r0 + j * ZROWS, ZROWS)], sem)
        return 0
    lax.fori_loop(0, n_z, wcopy, 0)

    def wdrain(j, _):
        pltpu.make_async_copy(acc_sh.at[pl.ds(r0, ZROWS)],
                              out_hbm.at[pl.ds(r0, ZROWS)], sem).wait()
        return 0
    lax.fori_loop(0, n_z, wdrain, 0)


def _edge_pipeline(h_hbm, s2, d2, acc_sh, sblk, dblk, rows_a, rows_b,
                   sem_ga, sem_gb, sid, process):
    row0 = sid * (PER_TILE // CHUNK)

    def refill(b):
        pltpu.sync_copy(s2.at[pl.ds(row0 + b * BCH, BCH)], sblk)
        pltpu.sync_copy(d2.at[pl.ds(row0 + b * BCH, BCH)], dblk)

    refill(0)
    pltpu.async_copy(h_hbm.at[sblk.at[0]], rows_a, sem_ga)

    def iter_body(c, _):
        def half(buf, sem_g, obuf, osem_g):
            jr = lax.rem(c, BCH)
            nxt = c + 1
            jn = lax.rem(nxt, BCH)
            pltpu.make_async_copy(h_hbm.at[sblk.at[jr]], buf, sem_g).wait()

            @pl.when((nxt < NCHUNK) & (jn != 0))
            def _():
                pltpu.async_copy(h_hbm.at[sblk.at[jn]], obuf, osem_g)

            process(buf, jr)
            pltpu.sync_copy(buf, acc_sh.at[dblk.at[jr]], add=True)

            @pl.when((nxt < NCHUNK) & (jn == 0))
            def _():
                refill(nxt // BCH)
                pltpu.async_copy(h_hbm.at[sblk.at[0]], obuf, osem_g)

        @pl.when(lax.rem(c, 2) == 0)
        def _():
            half(rows_a, sem_ga, rows_b, sem_gb)

        @pl.when(lax.rem(c, 2) == 1)
        def _():
            half(rows_b, sem_gb, rows_a, sem_ga)
        return 0
    lax.fori_loop(0, NCHUNK, iter_body, 0)


def _edge_pipeline3(h_hbm, s2, d2, acc_sh, sblk, dblk, bufs, sems, sid):
    row0 = sid * (PER_TILE // CHUNK)

    def refill(b):
        pltpu.sync_copy(s2.at[pl.ds(row0 + b * BCH, BCH)], sblk)
        pltpu.sync_copy(d2.at[pl.ds(row0 + b * BCH, BCH)], dblk)

    refill(0)
    pltpu.async_copy(h_hbm.at[sblk.at[0]], bufs[0], sems[0])

    def iter_body(c, _):
        def rot(buf, sem, buf1, sem1, buf2, sem2):
            jr = lax.rem(c, BCH)
            nxt = c + 1
            jn = lax.rem(nxt, BCH)
            pltpu.make_async_copy(h_hbm.at[sblk.at[jr]], buf, sem).wait()

            @pl.when(jr == 0)
            def _():
                pltpu.async_copy(h_hbm.at[sblk.at[1]], buf1, sem1)

            @pl.when((c + 2 < NCHUNK) & (jr <= BCH - 3))
            def _():
                pltpu.async_copy(h_hbm.at[sblk.at[jr + 2]], buf2, sem2)

            pltpu.sync_copy(buf, acc_sh.at[dblk.at[jr]], add=True)

            @pl.when((nxt < NCHUNK) & (jn == 0))
            def _():
                refill(nxt // BCH)
                pltpu.async_copy(h_hbm.at[sblk.at[0]], buf1, sem1)

        for par in range(3):
            @pl.when(lax.rem(c, 3) == par)
            def _(par=par):
                rot(bufs[par], sems[par],
                    bufs[(par + 1) % 3], sems[(par + 1) % 3],
                    bufs[(par + 2) % 3], sems[(par + 2) % 3])
        return 0
    lax.fori_loop(0, NCHUNK, iter_body, 0)


def _sage_sc_body(h_u2i, h_i2u, su2, du2, si2, di2, out_item, out_user,
                  acc_sh, sblk, dblk, rows_a, rows_b, rows_c,
                  sem_ga, sem_gb, sem_gc):
    cid = lax.axis_index("c")
    sid = lax.axis_index("s")

    r0, n_z = _zero_init(acc_sh, rows_a, sid, sem_ga, C)
    plsc.subcore_barrier()

    def run_rel(h_hbm, s2, d2, out_hbm):
        _edge_pipeline3(h_hbm, s2, d2, acc_sh, sblk, dblk,
                        (rows_a, rows_b, rows_c), (sem_ga, sem_gb, sem_gc), sid)
        plsc.subcore_barrier()
        _writeback(acc_sh, out_hbm, r0, n_z, sem_ga)

    @pl.when(cid == 0)
    def _():
        run_rel(h_u2i, su2, du2, out_item)

    @pl.when(cid == 1)
    def _():
        run_rel(h_i2u, si2, di2, out_user)


_sage_sc = pl.kernel(
    _sage_sc_body,
    out_type=(jax.ShapeDtypeStruct((N, C), jnp.float32),
              jax.ShapeDtypeStruct((N, C), jnp.float32)),
    mesh=plsc.VectorSubcoreMesh(core_axis_name="c", subcore_axis_name="s"),
    compiler_params=_SC_PARAMS,
    scratch_types=[
        pltpu.VMEM_SHARED((N, C), jnp.float32),
        pltpu.VMEM((BCH, CHUNK), jnp.int32),
        pltpu.VMEM((BCH, CHUNK), jnp.int32),
        pltpu.VMEM((CHUNK, C), jnp.float32),
        pltpu.VMEM((CHUNK, C), jnp.float32),
        pltpu.VMEM((CHUNK, C), jnp.float32),
        pltpu.SemaphoreType.DMA,
        pltpu.SemaphoreType.DMA,
        pltpu.SemaphoreType.DMA,
    ],
)


def _gat_sc_body(hsx_u2i, sdst_u2i, su2, du2,
                 hsx_i2u, sdst_i2u, si2, di2,
                 out_item, out_user,
                 acc_sh, sdst_sh, w_v, sblk, dblk, rows_a, rows_b, rows_c,
                 bb_a, bb_b, bb_c,
                 sem_a, sem_b, sem_c, bsem_a, bsem_b, bsem_c):
    cid = lax.axis_index("c")
    sid = lax.axis_index("s")

    r0, n_z = _zero_init(acc_sh, rows_a, sid, sem_a, CE)

    lanes = lax.iota(jnp.int32, LANES)
    col129 = jnp.full((LANES,), C + 1, jnp.int32)

    def run_rel(hsx_hbm, sdst_hbm, s2, d2, out_hbm):
        @pl.when(sid == 0)
        def _():
            pltpu.sync_copy(sdst_hbm, sdst_sh)
        plsc.subcore_barrier()

        row0 = sid * (PER_TILE // CHUNK)

        def refill(b):
            pltpu.sync_copy(s2.at[pl.ds(row0 + b * BCH, BCH)], sblk)
            pltpu.sync_copy(d2.at[pl.ds(row0 + b * BCH, BCH)], dblk)

        def start(jrow, rbuf, rsem, bbuf, bsem):
            pltpu.async_copy(hsx_hbm.at[sblk.at[jrow]], rbuf, rsem)
            pltpu.async_copy(sdst_sh.at[dblk.at[jrow]], bbuf, bsem)

        refill(0)
        start(0, rows_a, sem_a, bb_a, bsem_a)

        def iter_body(c, _):
            def rot(buf, sem, bbuf, bsem, r1, s1, bb1, bs1, r2, s2_, bb2, bs2):
                jr = lax.rem(c, BCH)
                nxt = c + 1
                jn = lax.rem(nxt, BCH)
                pltpu.make_async_copy(hsx_hbm.at[sblk.at[jr]], buf, sem).wait()
                pltpu.make_async_copy(sdst_sh.at[dblk.at[jr]], bbuf, bsem).wait()

                @pl.when(jr == 0)
                def _():
                    start(1, r1, s1, bb1, bs1)

                @pl.when((c + 2 < NCHUNK) & (jr <= BCH - 3))
                def _():
                    start(jr + 2, r2, s2_, bb2, bs2)

                for g in range(CHUNK // LANES):
                    e16 = lanes + g * LANES
                    a = plsc.load_gather(buf, [e16, col129])
                    b = bbuf[pl.ds(g * LANES, LANES)]
                    x = a + b
                    w_v[pl.ds(g * LANES, LANES)] = jnp.exp(jnp.where(x >= 0, x, 0.2 * x))

                @plsc.parallel_loop(0, CHUNK, step=1, unroll=4)
                def _(e):
                    wspl = plsc.load_gather(w_v, [jnp.full((LANES,), e, jnp.int32)])
                    for r in range(CE // LANES):
                        buf[e, pl.ds(r * LANES, LANES)] = buf[e, pl.ds(r * LANES, LANES)] * wspl

                pltpu.sync_copy(buf, acc_sh.at[dblk.at[jr]], add=True)

                @pl.when((nxt < NCHUNK) & (jn == 0))
                def _():
                    refill(nxt // BCH)
                    start(0, r1, s1, bb1, bs1)

            bufs = ((rows_a, sem_a, bb_a, bsem_a),
                    (rows_b, sem_b, bb_b, bsem_b),
                    (rows_c, sem_c, bb_c, bsem_c))
            for par in range(3):
                @pl.when(lax.rem(c, 3) == par)
                def _(par=par):
                    rot(*bufs[par], *bufs[(par + 1) % 3], *bufs[(par + 2) % 3])
            return 0
        lax.fori_loop(0, NCHUNK, iter_body, 0)
        plsc.subcore_barrier()
        _writeback(acc_sh, out_hbm, r0, n_z, sem_a)

    @pl.when(cid == 0)
    def _():
        run_rel(hsx_u2i, sdst_u2i, su2, du2, out_item)

    @pl.when(cid == 1)
    def _():
        run_rel(hsx_i2u, sdst_i2u, si2, di2, out_user)


_gat_sc = pl.kernel(
    _gat_sc_body,
    out_type=(jax.ShapeDtypeStruct((N, CE), jnp.float32),
              jax.ShapeDtypeStruct((N, CE), jnp.float32)),
    mesh=plsc.VectorSubcoreMesh(core_axis_name="c", subcore_axis_name="s"),
    compiler_params=_SC_PARAMS,
    scratch_types=[
        pltpu.VMEM_SHARED((N, CE), jnp.float32),
        pltpu.VMEM_SHARED((N,), jnp.float32),
        pltpu.VMEM((CHUNK,), jnp.float32),
        pltpu.VMEM((BCH, CHUNK), jnp.int32),
        pltpu.VMEM((BCH, CHUNK), jnp.int32),
        pltpu.VMEM((CHUNK, CE), jnp.float32),
        pltpu.VMEM((CHUNK, CE), jnp.float32),
        pltpu.VMEM((CHUNK, CE), jnp.float32),
        pltpu.VMEM((CHUNK,), jnp.float32),
        pltpu.VMEM((CHUNK,), jnp.float32),
        pltpu.VMEM((CHUNK,), jnp.float32),
        pltpu.SemaphoreType.DMA,
        pltpu.SemaphoreType.DMA,
        pltpu.SemaphoreType.DMA,
        pltpu.SemaphoreType.DMA,
        pltpu.SemaphoreType.DMA,
        pltpu.SemaphoreType.DMA,
    ],
)


def _ln(x, w, b):
    mu = x.mean(-1, keepdims=True)
    var = ((x - mu) ** 2).mean(-1, keepdims=True)
    return (x - mu) / jnp.sqrt(var + 1e-5) * w + b


BLK = 1000
_full = pl.BlockSpec((128, 128), lambda i: (0, 0))
_vec = pl.BlockSpec((1, 128), lambda i: (0, 0))
_rows = pl.BlockSpec((BLK, C), lambda i: (i, 0))
_rows_e = pl.BlockSpec((BLK, CE), lambda i: (i, 0))
_vec8 = pl.BlockSpec((128, 8), lambda i: (0, 0))


def _pre_h_body(xu, xi, wl_u, wl_i, h_u, h_i):
    h_u[...] = jnp.dot(xu[...], wl_u[...], preferred_element_type=jnp.float32)
    h_i[...] = jnp.dot(xi[...], wl_i[...], preferred_element_type=jnp.float32)


def _pre_h_tc(xu, xi, wl_u, wl_i):
    return pl.pallas_call(
        _pre_h_body,
        grid=(N // BLK,),
        in_specs=[_rows, _rows, _full, _full],
        out_specs=[_rows, _rows],
        out_shape=[jax.ShapeDtypeStruct((N, C), jnp.float32)] * 2,
    )(xu, xi, wl_u, wl_i)


def _mid_body(acc_i, acc_u, xu, xi, wr_u, wr_i, bl_u, bl_i,
              n0iw, n0ib, n0uw, n0ub,
              wsrc_u, wdst_u, asrc_u, adst_u,
              wsrc_i, wdst_i, asrc_i, adst_i,
              hsx_u, hsx_i, sdst_u, sdst_i):
    r_i = jnp.dot(xi[...], wr_u[...], preferred_element_type=jnp.float32) + bl_u[...]
    r_u = jnp.dot(xu[...], wr_i[...], preferred_element_type=jnp.float32) + bl_i[...]
    item1 = jax.nn.relu(_ln(acc_i[...] + r_i, n0iw[...], n0ib[...]))
    user1 = jax.nn.relu(_ln(acc_u[...] + r_u, n0uw[...], n0ub[...]))

    def gat_half(xs, xd, wsrc, wdst, asrc, adst, hsx, sdst):
        hs = jnp.dot(xs, wsrc[...], preferred_element_type=jnp.float32)
        hd = jnp.dot(xd, wdst[...], preferred_element_type=jnp.float32)
        ssrc8 = jnp.dot(hs, asrc[...], preferred_element_type=jnp.float32)
        hsx[...] = jnp.concatenate(
            [hs, jnp.ones((BLK, 1), jnp.float32), ssrc8[:, :1],
             jnp.zeros((BLK, CE - C - 2), jnp.float32)], axis=1)
        sdst[...] = jnp.dot(hd, adst[...], preferred_element_type=jnp.float32)[None, :, :1]

    gat_half(user1, item1, wsrc_u, wdst_u, asrc_u, adst_u, hsx_u, sdst_u)
    gat_half(item1, user1, wsrc_i, wdst_i, asrc_i, adst_i, hsx_i, sdst_i)


_sd1 = pl.BlockSpec((1, BLK, 1), lambda i: (i, 0, 0))


def _mid_tc(acc_i, acc_u, xu, xi, wr_u, wr_i, bl_u, bl_i, n0,
            wsrc_u, wdst_u, asrc_u, adst_u,
            wsrc_i, wdst_i, asrc_i, adst_i):
    return pl.pallas_call(
        _mid_body,
        grid=(N // BLK,),
        in_specs=[_rows, _rows, _rows, _rows, _full, _full, _vec, _vec,
                  _vec, _vec, _vec, _vec,
                  _full, _full, _vec8, _vec8,
                  _full, _full, _vec8, _vec8],
        out_specs=[_rows_e, _rows_e, _sd1, _sd1],
        out_shape=[jax.ShapeDtypeStruct((N, CE), jnp.float32),
                   jax.ShapeDtypeStruct((N, CE), jnp.float32),
                   jax.ShapeDtypeStruct((N // BLK, BLK, 1), jnp.float32),
                   jax.ShapeDtypeStruct((N // BLK, BLK, 1), jnp.float32)],
    )(acc_i, acc_u, xu, xi, wr_u, wr_i, bl_u, bl_i, *n0,
      wsrc_u, wdst_u, asrc_u, adst_u,
      wsrc_i, wdst_i, asrc_i, adst_i)


def _post_gat_body(acc_gi, acc_gu, b_u, b_i, n1iw, n1ib, n1uw, n1ub,
                   item2, user2):
    def fin(acc, b, w, bb):
        num = acc[:, :C]
        den = acc[:, C:C + 1]
        return jax.nn.relu(_ln(num / (den + 1e-16) + b[...], w[...], bb[...]))
    item2[...] = fin(acc_gi[...], b_u, n1iw, n1ib)
    user2[...] = fin(acc_gu[...], b_i, n1uw, n1ub)


def _post_gat_tc(acc_gi, acc_gu, b_u, b_i, n1):
    return pl.pallas_call(
        _post_gat_body,
        grid=(N // BLK,),
        in_specs=[_rows_e, _rows_e, _vec, _vec, _vec, _vec, _vec, _vec],
        out_specs=[_rows, _rows],
        out_shape=[jax.ShapeDtypeStruct((N, C), jnp.float32),
                   jax.ShapeDtypeStruct((N, C), jnp.float32)],
    )(acc_gi, acc_gu, b_u, b_i, *n1)


def kernel(x_user, x_item, edge_index_u2i, edge_index_i2u, params):
    p = params
    su2 = jnp.asarray(edge_index_u2i[0], jnp.int32).reshape(E // CHUNK, CHUNK)
    du2 = jnp.asarray(edge_index_u2i[1], jnp.int32).reshape(E // CHUNK, CHUNK)
    si2 = jnp.asarray(edge_index_i2u[0], jnp.int32).reshape(E // CHUNK, CHUNK)
    di2 = jnp.asarray(edge_index_i2u[1], jnp.int32).reshape(E // CHUNK, CHUNK)

    h_u2i, h_i2u = _pre_h_tc(x_user, x_item, p['sage_u2i_Wl'], p['sage_i2u_Wl'])
    acc_item, acc_user = _sage_sc(h_u2i, h_i2u, su2, du2, si2, di2)

    n0 = (p['norm0_item_w'][None, :], p['norm0_item_b'][None, :],
          p['norm0_user_w'][None, :], p['norm0_user_b'][None, :])
    hsx_u, hsx_i, sdst_u2, sdst_i2 = _mid_tc(
        acc_item, acc_user, x_user, x_item,
        p['sage_u2i_Wr'], p['sage_i2u_Wr'],
        p['sage_u2i_bl'][None, :], p['sage_i2u_bl'][None, :], n0,
        p['gat_u2i_Wsrc'], p['gat_u2i_Wdst'],
        jnp.broadcast_to(p['gat_u2i_asrc'][:, None], (C, 8)),
        jnp.broadcast_to(p['gat_u2i_adst'][:, None], (C, 8)),
        p['gat_i2u_Wsrc'], p['gat_i2u_Wdst'],
        jnp.broadcast_to(p['gat_i2u_asrc'][:, None], (C, 8)),
        jnp.broadcast_to(p['gat_i2u_adst'][:, None], (C, 8)))

    accg_item, accg_user = _gat_sc(
        hsx_u, sdst_u2.reshape(N), su2, du2,
        hsx_i, sdst_i2.reshape(N), si2, di2)

    n1 = (p['norm1_item_w'][None, :], p['norm1_item_b'][None, :],
          p['norm1_user_w'][None, :], p['norm1_user_b'][None, :])
    item2, user2 = _post_gat_tc(accg_item, accg_user,
                                p['gat_u2i_b'][None, :], p['gat_i2u_b'][None, :], n1)
    return (user2, item2)


# speedup vs baseline: 1.0711x; 1.0361x over previous
import jax
import jax.numpy as jnp
from jax import lax
from jax.experimental import pallas as pl
from jax.experimental.pallas import tpu as pltpu
from jax.experimental.pallas import tpu_sc as plsc

N = 10000
E = 320000
C = 128
CE = 144

NS = 16
LANES = 16
PER_TILE = E // NS
CHUNK = 80
NCHUNK = PER_TILE // CHUNK
BCH = 25
ROWS_MAIN = 624
ZROWS = 16

_SC_PARAMS = pltpu.CompilerParams(needs_layout_passes=False,
                                  use_tc_tiling_on_sc=False)


def _zero_init(acc_sh, zsrc, sid, sem, width):
    zv = jnp.zeros((LANES,), jnp.float32)

    def zrow(i, _):
        for r in range(width // LANES):
            zsrc[i, pl.ds(r * LANES, LANES)] = zv
        return 0
    lax.fori_loop(0, ZROWS, zrow, 0)

    r0 = sid * ROWS_MAIN
    n_z = jnp.where(sid == NS - 1, (N - (NS - 1) * ROWS_MAIN) // ZROWS,
                    ROWS_MAIN // ZROWS)

    zslice = zsrc.at[pl.ds(0, ZROWS)]

    def zcopy(j, _):
        pltpu.async_copy(zslice, acc_sh.at[pl.ds(r0 + j * ZROWS, ZROWS)], sem)
        return 0
    lax.fori_loop(0, n_z, zcopy, 0)

    def zdrain(j, _):
        pltpu.make_async_copy(zslice, acc_sh.at[pl.ds(r0, ZROWS)], sem).wait()
        return 0
    lax.fori_loop(0, n_z, zdrain, 0)
    return r0, n_z


def _writeback(acc_sh, out_hbm, r0, n_z, sem):
    def wcopy(j, _):
        pltpu.async_copy(acc_sh.at[pl.ds(r0 + j * ZROWS, ZROWS)],
                         out_hbm.at[pl.ds(r0 + j * ZROWS, ZROWS)], sem)
        return 0
    lax.fori_loop(0, n_z, wcopy, 0)

    def wdrain(j, _):
        pltpu.make_async_copy(acc_sh.at[pl.ds(r0, ZROWS)],
                              out_hbm.at[pl.ds(r0, ZROWS)], sem).wait()
        return 0
    lax.fori_loop(0, n_z, wdrain, 0)


def _edge_pipeline(h_hbm, s2, d2, acc_sh, sblk, dblk, rows_a, rows_b,
                   sem_ga, sem_gb, sid, process):
    row0 = sid * (PER_TILE // CHUNK)

    def refill(b):
        pltpu.sync_copy(s2.at[pl.ds(row0 + b * BCH, BCH)], sblk)
        pltpu.sync_copy(d2.at[pl.ds(row0 + b * BCH, BCH)], dblk)

    refill(0)
    pltpu.async_copy(h_hbm.at[sblk.at[0]], rows_a, sem_ga)

    def iter_body(c, _):
        def half(buf, sem_g, obuf, osem_g):
            jr = lax.rem(c, BCH)
            nxt = c + 1
            jn = lax.rem(nxt, BCH)
            pltpu.make_async_copy(h_hbm.at[sblk.at[jr]], buf, sem_g).wait()

            @pl.when((nxt < NCHUNK) & (jn != 0))
            def _():
                pltpu.async_copy(h_hbm.at[sblk.at[jn]], obuf, osem_g)

            process(buf, jr)
            pltpu.sync_copy(buf, acc_sh.at[dblk.at[jr]], add=True)

            @pl.when((nxt < NCHUNK) & (jn == 0))
            def _():
                refill(nxt // BCH)
                pltpu.async_copy(h_hbm.at[sblk.at[0]], obuf, osem_g)

        @pl.when(lax.rem(c, 2) == 0)
        def _():
            half(rows_a, sem_ga, rows_b, sem_gb)

        @pl.when(lax.rem(c, 2) == 1)
        def _():
            half(rows_b, sem_gb, rows_a, sem_ga)
        return 0
    lax.fori_loop(0, NCHUNK, iter_body, 0)


def _edge_pipeline3(h_hbm, s2, d2, acc_sh, sblk, dblk, bufs, sems, sid):
    row0 = sid * (PER_TILE // CHUNK)

    def refill(b):
        pltpu.sync_copy(s2.at[pl.ds(row0 + b * BCH, BCH)], sblk)
        pltpu.sync_copy(d2.at[pl.ds(row0 + b * BCH, BCH)], dblk)

    refill(0)
    pltpu.async_copy(h_hbm.at[sblk.at[0]], bufs[0], sems[0])

    def iter_body(c, _):
        def rot(buf, sem, buf1, sem1, buf2, sem2):
            jr = lax.rem(c, BCH)
            nxt = c + 1
            jn = lax.rem(nxt, BCH)
            pltpu.make_async_copy(h_hbm.at[sblk.at[jr]], buf, sem).wait()

            @pl.when(jr == 0)
            def _():
                pltpu.async_copy(h_hbm.at[sblk.at[1]], buf1, sem1)

            @pl.when((c + 2 < NCHUNK) & (jr <= BCH - 3))
            def _():
                pltpu.async_copy(h_hbm.at[sblk.at[jr + 2]], buf2, sem2)

            pltpu.sync_copy(buf, acc_sh.at[dblk.at[jr]], add=True)

            @pl.when((nxt < NCHUNK) & (jn == 0))
            def _():
                refill(nxt // BCH)
                pltpu.async_copy(h_hbm.at[sblk.at[0]], buf1, sem1)

        for par in range(3):
            @pl.when(lax.rem(c, 3) == par)
            def _(par=par):
                rot(bufs[par], sems[par],
                    bufs[(par + 1) % 3], sems[(par + 1) % 3],
                    bufs[(par + 2) % 3], sems[(par + 2) % 3])
        return 0
    lax.fori_loop(0, NCHUNK, iter_body, 0)


def _sage_sc_body(h_u2i, h_i2u, su2, du2, si2, di2, out_item, out_user,
                  acc_sh, sblk, dblk, rows_a, rows_b, rows_c,
                  sem_ga, sem_gb, sem_gc):
    cid = lax.axis_index("c")
    sid = lax.axis_index("s")

    r0, n_z = _zero_init(acc_sh, rows_a, sid, sem_ga, C)
    plsc.subcore_barrier()

    def run_rel(h_hbm, s2, d2, out_hbm):
        _edge_pipeline3(h_hbm, s2, d2, acc_sh, sblk, dblk,
                        (rows_a, rows_b, rows_c), (sem_ga, sem_gb, sem_gc), sid)
        plsc.subcore_barrier()
        _writeback(acc_sh, out_hbm, r0, n_z, sem_ga)

    @pl.when(cid == 0)
    def _():
        run_rel(h_u2i, su2, du2, out_item)

    @pl.when(cid == 1)
    def _():
        run_rel(h_i2u, si2, di2, out_user)


_sage_sc = pl.kernel(
    _sage_sc_body,
    out_type=(jax.ShapeDtypeStruct((N, C), jnp.float32),
              jax.ShapeDtypeStruct((N, C), jnp.float32)),
    mesh=plsc.VectorSubcoreMesh(core_axis_name="c", subcore_axis_name="s"),
    compiler_params=_SC_PARAMS,
    scratch_types=[
        pltpu.VMEM_SHARED((N, C), jnp.float32),
        pltpu.VMEM((BCH, CHUNK), jnp.int32),
        pltpu.VMEM((BCH, CHUNK), jnp.int32),
        pltpu.VMEM((CHUNK, C), jnp.float32),
        pltpu.VMEM((CHUNK, C), jnp.float32),
        pltpu.VMEM((CHUNK, C), jnp.float32),
        pltpu.SemaphoreType.DMA,
        pltpu.SemaphoreType.DMA,
        pltpu.SemaphoreType.DMA,
    ],
)


def _gat_sc_body(hsx_u2i, sdst_u2i, su2, du2,
                 hsx_i2u, sdst_i2u, si2, di2,
                 out_item, out_user,
                 acc_sh, sdst_sh, w_v, sblk, dblk, rows_a, rows_b, rows_c,
                 bb_a, bb_b, bb_c,
                 sem_a, sem_b, sem_c, bsem_a, bsem_b, bsem_c,
                 ssem_a, ssem_b, ssem_c):
    cid = lax.axis_index("c")
    sid = lax.axis_index("s")

    r0, n_z = _zero_init(acc_sh, rows_a, sid, sem_a, CE)

    lanes = lax.iota(jnp.int32, LANES)
    col129 = jnp.full((LANES,), C + 1, jnp.int32)

    def run_rel(hsx_hbm, sdst_hbm, s2, d2, out_hbm):
        @pl.when(sid == 0)
        def _():
            pltpu.sync_copy(sdst_hbm, sdst_sh)
        plsc.subcore_barrier()

        row0 = sid * (PER_TILE // CHUNK)

        def refill(b):
            pltpu.sync_copy(s2.at[pl.ds(row0 + b * BCH, BCH)], sblk)
            pltpu.sync_copy(d2.at[pl.ds(row0 + b * BCH, BCH)], dblk)

        def start(jrow, rbuf, rsem, bbuf, bsem):
            pltpu.async_copy(hsx_hbm.at[sblk.at[jrow]], rbuf, rsem)
            pltpu.async_copy(sdst_sh.at[dblk.at[jrow]], bbuf, bsem)

        refill(0)
        start(0, rows_a, sem_a, bb_a, bsem_a)

        def iter_body(c, _):
            def rot(buf, sem, bbuf, bsem, ssem,
                    r1, s1, bb1, bs1, _ss1, r2, s2_, bb2, bs2, ss2):
                jr = lax.rem(c, BCH)
                nxt = c + 1
                jn = lax.rem(nxt, BCH)
                pltpu.make_async_copy(hsx_hbm.at[sblk.at[jr]], buf, sem).wait()
                pltpu.make_async_copy(sdst_sh.at[dblk.at[jr]], bbuf, bsem).wait()

                for g in range(CHUNK // LANES):
                    e16 = lanes + g * LANES
                    a = plsc.load_gather(buf, [e16, col129])
                    b = bbuf[pl.ds(g * LANES, LANES)]
                    x = a + b
                    w_v[pl.ds(g * LANES, LANES)] = jnp.exp(jnp.where(x >= 0, x, 0.2 * x))

                @plsc.parallel_loop(0, CHUNK, step=1, unroll=4)
                def _(e):
                    wspl = plsc.load_gather(w_v, [jnp.full((LANES,), e, jnp.int32)])
                    for r in range(CE // LANES):
                        buf[e, pl.ds(r * LANES, LANES)] = buf[e, pl.ds(r * LANES, LANES)] * wspl

                @pl.when((c >= 1) & (jr != 0))
                def _():
                    pltpu.make_async_copy(r2, acc_sh.at[dblk.at[jr]], ss2).wait()

                pltpu.async_copy(buf, acc_sh.at[dblk.at[jr]], ssem, add=True)

                @pl.when(jr == 0)
                def _():
                    start(1, r1, s1, bb1, bs1)

                @pl.when((c + 2 < NCHUNK) & (jr <= BCH - 3))
                def _():
                    start(jr + 2, r2, s2_, bb2, bs2)

                @pl.when((nxt < NCHUNK) & (jn == 0))
                def _():
                    pltpu.make_async_copy(buf, acc_sh.at[dblk.at[jr]], ssem).wait()
                    refill(nxt // BCH)
                    start(0, r1, s1, bb1, bs1)

            bufs = ((rows_a, sem_a, bb_a, bsem_a, ssem_a),
                    (rows_b, sem_b, bb_b, bsem_b, ssem_b),
                    (rows_c, sem_c, bb_c, bsem_c, ssem_c))
            for par in range(3):
                @pl.when(lax.rem(c, 3) == par)
                def _(par=par):
                    rot(*bufs[par], *bufs[(par + 1) % 3], *bufs[(par + 2) % 3])
            return 0
        lax.fori_loop(0, NCHUNK, iter_body, 0)
        pltpu.make_async_copy(rows_a, acc_sh.at[dblk.at[0]],
                              (ssem_a, ssem_b, ssem_c)[(NCHUNK - 1) % 3]).wait()
        plsc.subcore_barrier()
        _writeback(acc_sh, out_hbm, r0, n_z, sem_a)

    @pl.when(cid == 0)
    def _():
        run_rel(hsx_u2i, sdst_u2i, su2, du2, out_item)

    @pl.when(cid == 1)
    def _():
        run_rel(hsx_i2u, sdst_i2u, si2, di2, out_user)


_gat_sc = pl.kernel(
    _gat_sc_body,
    out_type=(jax.ShapeDtypeStruct((N, CE), jnp.float32),
              jax.ShapeDtypeStruct((N, CE), jnp.float32)),
    mesh=plsc.VectorSubcoreMesh(core_axis_name="c", subcore_axis_name="s"),
    compiler_params=_SC_PARAMS,
    scratch_types=[
        pltpu.VMEM_SHARED((N, CE), jnp.float32),
        pltpu.VMEM_SHARED((N,), jnp.float32),
        pltpu.VMEM((CHUNK,), jnp.float32),
        pltpu.VMEM((BCH, CHUNK), jnp.int32),
        pltpu.VMEM((BCH, CHUNK), jnp.int32),
        pltpu.VMEM((CHUNK, CE), jnp.float32),
        pltpu.VMEM((CHUNK, CE), jnp.float32),
        pltpu.VMEM((CHUNK, CE), jnp.float32),
        pltpu.VMEM((CHUNK,), jnp.float32),
        pltpu.VMEM((CHUNK,), jnp.float32),
        pltpu.VMEM((CHUNK,), jnp.float32),
        pltpu.SemaphoreType.DMA,
        pltpu.SemaphoreType.DMA,
        pltpu.SemaphoreType.DMA,
        pltpu.SemaphoreType.DMA,
        pltpu.SemaphoreType.DMA,
        pltpu.SemaphoreType.DMA,
        pltpu.SemaphoreType.DMA,
        pltpu.SemaphoreType.DMA,
        pltpu.SemaphoreType.DMA,
    ],
)


def _ln(x, w, b):
    mu = x.mean(-1, keepdims=True)
    var = ((x - mu) ** 2).mean(-1, keepdims=True)
    return (x - mu) / jnp.sqrt(var + 1e-5) * w + b


BLK = 1000
_full = pl.BlockSpec((128, 128), lambda i: (0, 0))
_vec = pl.BlockSpec((1, 128), lambda i: (0, 0))
_rows = pl.BlockSpec((BLK, C), lambda i: (i, 0))
_rows_e = pl.BlockSpec((BLK, CE), lambda i: (i, 0))
_vec8 = pl.BlockSpec((128, 8), lambda i: (0, 0))


def _pre_h_body(xu, xi, wl_u, wl_i, h_u, h_i):
    h_u[...] = jnp.dot(xu[...], wl_u[...], preferred_element_type=jnp.float32)
    h_i[...] = jnp.dot(xi[...], wl_i[...], preferred_element_type=jnp.float32)


def _pre_h_tc(xu, xi, wl_u, wl_i):
    return pl.pallas_call(
        _pre_h_body,
        grid=(N // BLK,),
        in_specs=[_rows, _rows, _full, _full],
        out_specs=[_rows, _rows],
        out_shape=[jax.ShapeDtypeStruct((N, C), jnp.float32)] * 2,
    )(xu, xi, wl_u, wl_i)


def _mid_body(acc_i, acc_u, xu, xi, wr_u, wr_i, bl_u, bl_i,
              n0iw, n0ib, n0uw, n0ub,
              wsrc_u, wdst_u, asrc_u, adst_u,
              wsrc_i, wdst_i, asrc_i, adst_i,
              hsx_u, hsx_i, sdst_u, sdst_i):
    r_i = jnp.dot(xi[...], wr_u[...], preferred_element_type=jnp.float32) + bl_u[...]
    r_u = jnp.dot(xu[...], wr_i[...], preferred_element_type=jnp.float32) + bl_i[...]
    item1 = jax.nn.relu(_ln(acc_i[...] + r_i, n0iw[...], n0ib[...]))
    user1 = jax.nn.relu(_ln(acc_u[...] + r_u, n0uw[...], n0ub[...]))

    def gat_half(xs, xd, wsrc, wdst, asrc, adst, hsx, sdst):
        hs = jnp.dot(xs, wsrc[...], preferred_element_type=jnp.float32)
        hd = jnp.dot(xd, wdst[...], preferred_element_type=jnp.float32)
        ssrc8 = jnp.dot(hs, asrc[...], preferred_element_type=jnp.float32)
        hsx[...] = jnp.concatenate(
            [hs, jnp.ones((BLK, 1), jnp.float32), ssrc8[:, :1],
             jnp.zeros((BLK, CE - C - 2), jnp.float32)], axis=1)
        sdst[...] = jnp.dot(hd, adst[...], preferred_element_type=jnp.float32)[None, :, :1]

    gat_half(user1, item1, wsrc_u, wdst_u, asrc_u, adst_u, hsx_u, sdst_u)
    gat_half(item1, user1, wsrc_i, wdst_i, asrc_i, adst_i, hsx_i, sdst_i)


_sd1 = pl.BlockSpec((1, BLK, 1), lambda i: (i, 0, 0))


def _mid_tc(acc_i, acc_u, xu, xi, wr_u, wr_i, bl_u, bl_i, n0,
            wsrc_u, wdst_u, asrc_u, adst_u,
            wsrc_i, wdst_i, asrc_i, adst_i):
    return pl.pallas_call(
        _mid_body,
        grid=(N // BLK,),
        in_specs=[_rows, _rows, _rows, _rows, _full, _full, _vec, _vec,
                  _vec, _vec, _vec, _vec,
                  _full, _full, _vec8, _vec8,
                  _full, _full, _vec8, _vec8],
        out_specs=[_rows_e, _rows_e, _sd1, _sd1],
        out_shape=[jax.ShapeDtypeStruct((N, CE), jnp.float32),
                   jax.ShapeDtypeStruct((N, CE), jnp.float32),
                   jax.ShapeDtypeStruct((N // BLK, BLK, 1), jnp.float32),
                   jax.ShapeDtypeStruct((N // BLK, BLK, 1), jnp.float32)],
    )(acc_i, acc_u, xu, xi, wr_u, wr_i, bl_u, bl_i, *n0,
      wsrc_u, wdst_u, asrc_u, adst_u,
      wsrc_i, wdst_i, asrc_i, adst_i)


def _post_gat_body(acc_gi, acc_gu, b_u, b_i, n1iw, n1ib, n1uw, n1ub,
                   item2, user2):
    def fin(acc, b, w, bb):
        num = acc[:, :C]
        den = acc[:, C:C + 1]
        return jax.nn.relu(_ln(num / (den + 1e-16) + b[...], w[...], bb[...]))
    item2[...] = fin(acc_gi[...], b_u, n1iw, n1ib)
    user2[...] = fin(acc_gu[...], b_i, n1uw, n1ub)


def _post_gat_tc(acc_gi, acc_gu, b_u, b_i, n1):
    return pl.pallas_call(
        _post_gat_body,
        grid=(N // BLK,),
        in_specs=[_rows_e, _rows_e, _vec, _vec, _vec, _vec, _vec, _vec],
        out_specs=[_rows, _rows],
        out_shape=[jax.ShapeDtypeStruct((N, C), jnp.float32),
                   jax.ShapeDtypeStruct((N, C), jnp.float32)],
    )(acc_gi, acc_gu, b_u, b_i, *n1)


def kernel(x_user, x_item, edge_index_u2i, edge_index_i2u, params):
    p = params
    su2 = jnp.asarray(edge_index_u2i[0], jnp.int32).reshape(E // CHUNK, CHUNK)
    du2 = jnp.asarray(edge_index_u2i[1], jnp.int32).reshape(E // CHUNK, CHUNK)
    si2 = jnp.asarray(edge_index_i2u[0], jnp.int32).reshape(E // CHUNK, CHUNK)
    di2 = jnp.asarray(edge_index_i2u[1], jnp.int32).reshape(E // CHUNK, CHUNK)

    h_u2i, h_i2u = _pre_h_tc(x_user, x_item, p['sage_u2i_Wl'], p['sage_i2u_Wl'])
    acc_item, acc_user = _sage_sc(h_u2i, h_i2u, su2, du2, si2, di2)

    n0 = (p['norm0_item_w'][None, :], p['norm0_item_b'][None, :],
          p['norm0_user_w'][None, :], p['norm0_user_b'][None, :])
    hsx_u, hsx_i, sdst_u2, sdst_i2 = _mid_tc(
        acc_item, acc_user, x_user, x_item,
        p['sage_u2i_Wr'], p['sage_i2u_Wr'],
        p['sage_u2i_bl'][None, :], p['sage_i2u_bl'][None, :], n0,
        p['gat_u2i_Wsrc'], p['gat_u2i_Wdst'],
        jnp.broadcast_to(p['gat_u2i_asrc'][:, None], (C, 8)),
        jnp.broadcast_to(p['gat_u2i_adst'][:, None], (C, 8)),
        p['gat_i2u_Wsrc'], p['gat_i2u_Wdst'],
        jnp.broadcast_to(p['gat_i2u_asrc'][:, None], (C, 8)),
        jnp.broadcast_to(p['gat_i2u_adst'][:, None], (C, 8)))

    accg_item, accg_user = _gat_sc(
        hsx_u, sdst_u2.reshape(N), su2, du2,
        hsx_i, sdst_i2.reshape(N), si2, di2)

    n1 = (p['norm1_item_w'][None, :], p['norm1_item_b'][None, :],
          p['norm1_user_w'][None, :], p['norm1_user_b'][None, :])
    item2, user2 = _post_gat_tc(accg_item, accg_user,
                                p['gat_u2i_b'][None, :], p['gat_i2u_b'][None, :], n1)
    return (user2, item2)
